# baseline (speedup 1.0000x reference)
import jax
import jax.numpy as jnp
from jax import lax
from jax.experimental import pallas as pl
from jax.experimental.pallas import tpu as pltpu
from jax.experimental.pallas import tpu_sc as plsc

_N = 10000
_E = 320000
_D = 128
_NC = 2
_NS = 16
_NW = _NC * _NS
_C = 80
_T = _E // _C
_TPW = _T // _NW
_RPT = (_N // _NS) // 8 * 8
_RTAIL = _N - _NS * _RPT
_SB = 208
_DW = 16

_mesh = plsc.VectorSubcoreMesh(core_axis_name="c", subcore_axis_name="s")


def _sc_agg_body(h_hbm, src_hbm, dst_hbm, zf_hbm, agg_out,
                 src_i, dst_i, rows, sbuf, agg_sh,
                 gs0, gs1, ss0, ss1):
    gsems = (gs0, gs1)
    ssems = (ss0, ss1)
    c = lax.axis_index("c")
    s = lax.axis_index("s")
    wid = s * _NC + c
    wbase = wid * (_E // _NW)
    tb = _NS * _RPT

    pltpu.sync_copy(zf_hbm, sbuf)
    for k in range(_RPT // _SB):
        pltpu.sync_copy(sbuf, agg_sh.at[pl.ds(s * _RPT + k * _SB, _SB)])

    @pl.when(s == 0)
    def _():
        pltpu.sync_copy(sbuf.at[pl.ds(0, _RTAIL)], agg_sh.at[pl.ds(tb, _RTAIL)])

    def stage(i, b):
        base = wbase + i * _C
        pltpu.sync_copy(src_hbm.at[pl.ds(base, _C)], src_i.at[b])
        pltpu.sync_copy(dst_hbm.at[pl.ds(base, _C)], dst_i.at[b])
        pltpu.async_copy(h_hbm.at[src_i.at[b]], rows.at[b], gsems[b])

    def wait_g(b):
        pltpu.make_async_copy(h_hbm.at[src_i.at[b]], rows.at[b],
                              gsems[b]).wait()

    def scat(b):
        pltpu.async_copy(rows.at[b], agg_sh.at[dst_i.at[b]], ssems[b],
                         add=True)

    def wait_s(b):
        pltpu.make_async_copy(rows.at[b], agg_sh.at[dst_i.at[b]],
                              ssems[b]).wait()

    stage(0, 0)
    plsc.subcore_barrier()

    wait_g(0); scat(0); stage(1, 1)
    wait_g(1); scat(1); wait_s(0); stage(2, 0)

    def duo(j, carry):
        for k in range(2):
            b = k
            i = 2 + 2 * j + k
            wait_g(b)
            scat(b)
            wait_s(1 - b)
            stage(i + 1, 1 - b)
        return carry

    lax.fori_loop(0, (_TPW - 3) // 2, duo, 0)
    wait_g(0); scat(0); wait_s(1)
    wait_s(0)
    plsc.subcore_barrier()

    for k in range(_RPT // _SB):
        off = s * _RPT + k * _SB
        pltpu.sync_copy(agg_sh.at[pl.ds(off, _SB)], sbuf)
        pltpu.sync_copy(sbuf, agg_out.at[c].at[pl.ds(off, _SB)])

    @pl.when(s == 0)
    def _():
        pltpu.sync_copy(agg_sh.at[pl.ds(tb, _RTAIL)], sbuf.at[pl.ds(0, _RTAIL)])
        pltpu.sync_copy(sbuf.at[pl.ds(0, _RTAIL)],
                        agg_out.at[c].at[pl.ds(tb, _RTAIL)])


_sc_agg = pl.kernel(
    _sc_agg_body,
    out_type=[jax.ShapeDtypeStruct((_NC, _N, _D), jnp.float32)],
    mesh=_mesh,
    scratch_types=[
        pltpu.VMEM((2, _C), jnp.int32),
        pltpu.VMEM((2, _C), jnp.int32),
        pltpu.VMEM((2, _C, _D), jnp.float32),
        pltpu.VMEM((_SB, _D), jnp.float32),
        pltpu.VMEM_SHARED((_N, _D), jnp.float32),
        pltpu.SemaphoreType.DMA,
        pltpu.SemaphoreType.DMA,
        pltpu.SemaphoreType.DMA,
        pltpu.SemaphoreType.DMA,
    ],
)


def _sc_deg_body(dst_hbm, zf_hbm, ones_hbm, deg_out,
                 dst_i, ones_v, sbuf, deg_sh, ss0, ss1):
    ssems = (ss0, ss1)
    c = lax.axis_index("c")
    s = lax.axis_index("s")
    wid = s * _NC + c
    wbase = wid * (_E // _NW)
    tb = _NS * _RPT

    pltpu.sync_copy(zf_hbm, sbuf)
    for k in range(_RPT // _SB):
        pltpu.sync_copy(sbuf, deg_sh.at[pl.ds(s * _RPT + k * _SB, _SB)])
    pltpu.sync_copy(ones_hbm, ones_v)

    @pl.when(s == 0)
    def _():
        pltpu.sync_copy(sbuf.at[pl.ds(0, _RTAIL)], deg_sh.at[pl.ds(tb, _RTAIL)])

    def stage(i, b):
        pltpu.sync_copy(dst_hbm.at[pl.ds(wbase + i * _C, _C)], dst_i.at[b])

    def scat(b):
        pltpu.async_copy(ones_v, deg_sh.at[dst_i.at[b]], ssems[b], add=True)

    def wait_s(b):
        pltpu.make_async_copy(ones_v, deg_sh.at[dst_i.at[b]], ssems[b]).wait()

    stage(0, 0)
    plsc.subcore_barrier()

    scat(0); stage(1, 1)
    scat(1); wait_s(0); stage(2, 0)

    def duo(j, carry):
        for k in range(2):
            b = k
            i = 2 + 2 * j + k
            scat(b)
            wait_s(1 - b)
            stage(i + 1, 1 - b)
        return carry

    lax.fori_loop(0, (_TPW - 3) // 2, duo, 0)
    scat(0); wait_s(1)
    wait_s(0)
    plsc.subcore_barrier()

    for k in range(_RPT // _SB):
        off = s * _RPT + k * _SB
        pltpu.sync_copy(deg_sh.at[pl.ds(off, _SB)], sbuf)
        pltpu.sync_copy(sbuf, deg_out.at[c].at[pl.ds(off, _SB)])

    @pl.when(s == 0)
    def _():
        pltpu.sync_copy(deg_sh.at[pl.ds(tb, _RTAIL)], sbuf.at[pl.ds(0, _RTAIL)])
        pltpu.sync_copy(sbuf.at[pl.ds(0, _RTAIL)],
                        deg_out.at[c].at[pl.ds(tb, _RTAIL)])


_sc_deg = pl.kernel(
    _sc_deg_body,
    out_type=[jax.ShapeDtypeStruct((_NC, _N, _D), jnp.float32)],
    mesh=_mesh,
    scratch_types=[
        pltpu.VMEM((2, _C), jnp.int32),
        pltpu.VMEM((_C, _D), jnp.float32),
        pltpu.VMEM((_SB, _D), jnp.float32),
        pltpu.VMEM_SHARED((_N, _D), jnp.float32),
        pltpu.SemaphoreType.DMA,
        pltpu.SemaphoreType.DMA,
    ],
)


def _mm_t(a, w):
    return lax.dot_general(a, w, (((1,), (1,)), ((), ())),
                           preferred_element_type=jnp.float32)


def _tc_bn_body(agg_ref, deg_ref, h_ref, wn_ref, ws_ref, b_ref, g_ref,
                be_ref, out_ref):
    dd = deg_ref[0] + deg_ref[1]
    invd = 1.0 / jnp.maximum(dd[:, 0:1], 1.0)
    agg = (agg_ref[0] + agg_ref[1]) * invd
    y = _mm_t(agg, wn_ref[...]) + _mm_t(h_ref[...], ws_ref[...]) + b_ref[...]
    m = jnp.mean(y, axis=0, keepdims=True)
    v = jnp.mean((y - m) ** 2, axis=0, keepdims=True)
    yn = (y - m) * lax.rsqrt(v + 1e-5) * g_ref[...] + be_ref[...]
    out_ref[...] = jnp.where(yn >= 0, yn, 0.1 * yn)


def _tc_final_body(agg_ref, deg_ref, h_ref, wn_ref, ws_ref, b_ref, out_ref):
    dd = deg_ref[0] + deg_ref[1]
    invd = 1.0 / jnp.maximum(dd[:, 0:1], 1.0)
    agg = (agg_ref[0] + agg_ref[1]) * invd
    out_ref[...] = (_mm_t(agg, wn_ref[...]) + _mm_t(h_ref[...], ws_ref[...])
                    + b_ref[...])


def _tc_bn(agg, deg, h, wn, ws, b, g, be):
    return pl.pallas_call(
        _tc_bn_body,
        out_shape=jax.ShapeDtypeStruct((_N, _D), jnp.float32),
    )(agg, deg, h, wn, ws, b, g, be)


def _tc_final(agg, deg, h, wn, ws, b):
    return pl.pallas_call(
        _tc_final_body,
        out_shape=jax.ShapeDtypeStruct((_N, _D), jnp.float32),
    )(agg, deg, h, wn, ws, b)


def kernel(x, edge_index, W1n, W1s, b1, g1, be1, W2n, W2s, b2, g2, be2,
           W3n, W3s, b3):
    src = edge_index[0].astype(jnp.int32)
    dst = edge_index[1].astype(jnp.int32)
    zf = jnp.zeros((_SB, _D), jnp.float32)
    ones = jnp.ones((_C, _D), jnp.float32)
    b1r, g1r, be1r = b1.reshape(1, -1), g1.reshape(1, -1), be1.reshape(1, -1)
    b2r, g2r, be2r = b2.reshape(1, -1), g2.reshape(1, -1), be2.reshape(1, -1)
    b3r = b3.reshape(1, -1)

    (deg,) = _sc_deg(dst, zf, ones)
    (agg1,) = _sc_agg(x, src, dst, zf)
    h1 = _tc_bn(agg1, deg, x, W1n, W1s, b1r, g1r, be1r)
    (agg2,) = _sc_agg(h1, src, dst, zf)
    h2 = _tc_bn(agg2, deg, h1, W2n, W2s, b2r, g2r, be2r)
    (agg3,) = _sc_agg(h2, src, dst, zf)
    return _tc_final(agg3, deg, h2, W3n, W3s, b3r)


# speedup vs baseline: 1.3217x; 1.3217x over previous
import jax
import jax.numpy as jnp
from jax import lax
from jax.experimental import pallas as pl
from jax.experimental.pallas import tpu as pltpu
from jax.experimental.pallas import tpu_sc as plsc

_N = 10000
_E = 320000
_D = 128
_NC = 2
_NS = 16
_NW = _NC * _NS
_C = 80
_T = _E // _C
_TPW = _T // _NW
_RPT = (_N // _NS) // 8 * 8
_RTAIL = _N - _NS * _RPT
_SB = 208
_DW = 16

_mesh = plsc.VectorSubcoreMesh(core_axis_name="c", subcore_axis_name="s")


def _sc_agg_body(h_hbm, src_hbm, dst_hbm, zf_hbm, agg_out,
                 src_i, dst_i, rows, sbuf, agg_sh,
                 gs0, gs1, ss0, ss1):
    gsems = (gs0, gs1)
    ssems = (ss0, ss1)
    c = lax.axis_index("c")
    s = lax.axis_index("s")
    wid = s * _NC + c
    wbase = wid * (_E // _NW)
    tb = _NS * _RPT

    pltpu.sync_copy(zf_hbm, sbuf)
    for k in range(_RPT // _SB):
        pltpu.sync_copy(sbuf, agg_sh.at[pl.ds(s * _RPT + k * _SB, _SB)])

    @pl.when(s == 0)
    def _():
        pltpu.sync_copy(sbuf.at[pl.ds(0, _RTAIL)], agg_sh.at[pl.ds(tb, _RTAIL)])

    def stage(i, b):
        base = wbase + i * _C
        pltpu.sync_copy(src_hbm.at[pl.ds(base, _C)], src_i.at[b])
        pltpu.sync_copy(dst_hbm.at[pl.ds(base, _C)], dst_i.at[b])
        pltpu.async_copy(h_hbm.at[src_i.at[b]], rows.at[b], gsems[b])

    def wait_g(b):
        pltpu.make_async_copy(h_hbm.at[src_i.at[b]], rows.at[b],
                              gsems[b]).wait()

    def scat(b):
        pltpu.async_copy(rows.at[b], agg_sh.at[dst_i.at[b]], ssems[b],
                         add=True)

    def wait_s(b):
        pltpu.make_async_copy(rows.at[b], agg_sh.at[dst_i.at[b]],
                              ssems[b]).wait()

    stage(0, 0)
    plsc.subcore_barrier()

    def duo(j, carry):
        for k in range(2):
            b = k
            i = 2 * j + k
            stage(i + 1, 1 - b)
            wait_g(b)
            pltpu.sync_copy(rows.at[b], agg_sh.at[dst_i.at[b]], add=True)
        return carry

    lax.fori_loop(0, (_TPW - 1) // 2, duo, 0)
    wait_g(0)
    pltpu.sync_copy(rows.at[0], agg_sh.at[dst_i.at[0]], add=True)
    plsc.subcore_barrier()

    for k in range(_RPT // _SB):
        off = s * _RPT + k * _SB
        pltpu.sync_copy(agg_sh.at[pl.ds(off, _SB)], sbuf)
        pltpu.sync_copy(sbuf, agg_out.at[c].at[pl.ds(off, _SB)])

    @pl.when(s == 0)
    def _():
        pltpu.sync_copy(agg_sh.at[pl.ds(tb, _RTAIL)], sbuf.at[pl.ds(0, _RTAIL)])
        pltpu.sync_copy(sbuf.at[pl.ds(0, _RTAIL)],
                        agg_out.at[c].at[pl.ds(tb, _RTAIL)])


_sc_agg = pl.kernel(
    _sc_agg_body,
    out_type=[jax.ShapeDtypeStruct((_NC, _N, _D), jnp.float32)],
    mesh=_mesh,
    scratch_types=[
        pltpu.VMEM((2, _C), jnp.int32),
        pltpu.VMEM((2, _C), jnp.int32),
        pltpu.VMEM((2, _C, _D), jnp.float32),
        pltpu.VMEM((_SB, _D), jnp.float32),
        pltpu.VMEM_SHARED((_N, _D), jnp.float32),
        pltpu.SemaphoreType.DMA,
        pltpu.SemaphoreType.DMA,
        pltpu.SemaphoreType.DMA,
        pltpu.SemaphoreType.DMA,
    ],
)


def _sc_deg_body(dst_hbm, zf_hbm, ones_hbm, deg_out,
                 dst_i, ones_v, sbuf, deg_sh, ss0, ss1):
    ssems = (ss0, ss1)
    c = lax.axis_index("c")
    s = lax.axis_index("s")
    wid = s * _NC + c
    wbase = wid * (_E // _NW)
    tb = _NS * _RPT

    pltpu.sync_copy(zf_hbm, sbuf)
    for k in range(_RPT // _SB):
        pltpu.sync_copy(sbuf, deg_sh.at[pl.ds(s * _RPT + k * _SB, _SB)])
    pltpu.sync_copy(ones_hbm, ones_v)

    @pl.when(s == 0)
    def _():
        pltpu.sync_copy(sbuf.at[pl.ds(0, _RTAIL)], deg_sh.at[pl.ds(tb, _RTAIL)])

    def stage(i, b):
        pltpu.sync_copy(dst_hbm.at[pl.ds(wbase + i * _C, _C)], dst_i.at[b])

    def scat(b):
        pltpu.async_copy(ones_v, deg_sh.at[dst_i.at[b]], ssems[b], add=True)

    def wait_s(b):
        pltpu.make_async_copy(ones_v, deg_sh.at[dst_i.at[b]], ssems[b]).wait()

    stage(0, 0)
    plsc.subcore_barrier()

    scat(0); stage(1, 1)
    scat(1); wait_s(0); stage(2, 0)

    def duo(j, carry):
        for k in range(2):
            b = k
            i = 2 + 2 * j + k
            scat(b)
            wait_s(1 - b)
            stage(i + 1, 1 - b)
        return carry

    lax.fori_loop(0, (_TPW - 3) // 2, duo, 0)
    scat(0); wait_s(1)
    wait_s(0)
    plsc.subcore_barrier()

    for k in range(_RPT // _SB):
        off = s * _RPT + k * _SB
        pltpu.sync_copy(deg_sh.at[pl.ds(off, _SB)], sbuf)
        pltpu.sync_copy(sbuf, deg_out.at[c].at[pl.ds(off, _SB)])

    @pl.when(s == 0)
    def _():
        pltpu.sync_copy(deg_sh.at[pl.ds(tb, _RTAIL)], sbuf.at[pl.ds(0, _RTAIL)])
        pltpu.sync_copy(sbuf.at[pl.ds(0, _RTAIL)],
                        deg_out.at[c].at[pl.ds(tb, _RTAIL)])


_sc_deg = pl.kernel(
    _sc_deg_body,
    out_type=[jax.ShapeDtypeStruct((_NC, _N, _D), jnp.float32)],
    mesh=_mesh,
    scratch_types=[
        pltpu.VMEM((2, _C), jnp.int32),
        pltpu.VMEM((_C, _D), jnp.float32),
        pltpu.VMEM((_SB, _D), jnp.float32),
        pltpu.VMEM_SHARED((_N, _D), jnp.float32),
        pltpu.SemaphoreType.DMA,
        pltpu.SemaphoreType.DMA,
    ],
)


def _mm_t(a, w):
    return lax.dot_general(a, w, (((1,), (1,)), ((), ())),
                           preferred_element_type=jnp.float32)


def _tc_bn_body(agg_ref, deg_ref, h_ref, wn_ref, ws_ref, b_ref, g_ref,
                be_ref, out_ref):
    dd = deg_ref[0] + deg_ref[1]
    invd = 1.0 / jnp.maximum(dd[:, 0:1], 1.0)
    agg = (agg_ref[0] + agg_ref[1]) * invd
    y = _mm_t(agg, wn_ref[...]) + _mm_t(h_ref[...], ws_ref[...]) + b_ref[...]
    m = jnp.mean(y, axis=0, keepdims=True)
    v = jnp.mean((y - m) ** 2, axis=0, keepdims=True)
    yn = (y - m) * lax.rsqrt(v + 1e-5) * g_ref[...] + be_ref[...]
    out_ref[...] = jnp.where(yn >= 0, yn, 0.1 * yn)


def _tc_final_body(agg_ref, deg_ref, h_ref, wn_ref, ws_ref, b_ref, out_ref):
    dd = deg_ref[0] + deg_ref[1]
    invd = 1.0 / jnp.maximum(dd[:, 0:1], 1.0)
    agg = (agg_ref[0] + agg_ref[1]) * invd
    out_ref[...] = (_mm_t(agg, wn_ref[...]) + _mm_t(h_ref[...], ws_ref[...])
                    + b_ref[...])


def _tc_bn(agg, deg, h, wn, ws, b, g, be):
    return pl.pallas_call(
        _tc_bn_body,
        out_shape=jax.ShapeDtypeStruct((_N, _D), jnp.float32),
    )(agg, deg, h, wn, ws, b, g, be)


def _tc_final(agg, deg, h, wn, ws, b):
    return pl.pallas_call(
        _tc_final_body,
        out_shape=jax.ShapeDtypeStruct((_N, _D), jnp.float32),
    )(agg, deg, h, wn, ws, b)


def kernel(x, edge_index, W1n, W1s, b1, g1, be1, W2n, W2s, b2, g2, be2,
           W3n, W3s, b3):
    src = edge_index[0].astype(jnp.int32)
    dst = edge_index[1].astype(jnp.int32)
    zf = jnp.zeros((_SB, _D), jnp.float32)
    ones = jnp.ones((_C, _D), jnp.float32)
    b1r, g1r, be1r = b1.reshape(1, -1), g1.reshape(1, -1), be1.reshape(1, -1)
    b2r, g2r, be2r = b2.reshape(1, -1), g2.reshape(1, -1), be2.reshape(1, -1)
    b3r = b3.reshape(1, -1)

    (deg,) = _sc_deg(dst, zf, ones)
    (agg1,) = _sc_agg(x, src, dst, zf)
    h1 = _tc_bn(agg1, deg, x, W1n, W1s, b1r, g1r, be1r)
    (agg2,) = _sc_agg(h1, src, dst, zf)
    h2 = _tc_bn(agg2, deg, h1, W2n, W2s, b2r, g2r, be2r)
    (agg3,) = _sc_agg(h2, src, dst, zf)
    return _tc_final(agg3, deg, h2, W3n, W3s, b3r)


# speedup vs baseline: 1.8290x; 1.3838x over previous
import jax
import jax.numpy as jnp
from jax import lax
from jax.experimental import pallas as pl
from jax.experimental.pallas import tpu as pltpu
from jax.experimental.pallas import tpu_sc as plsc

_N = 10000
_E = 320000
_D = 128
_NC = 2
_NS = 16
_NW = _NC * _NS
_C = 80
_T = _E // _C
_TPW = _T // _NW
_RPT = (_N // _NS) // 8 * 8
_RTAIL = _N - _NS * _RPT
_SB = 208
_DW = 16

_mesh = plsc.VectorSubcoreMesh(core_axis_name="c", subcore_axis_name="s")


def _sc_agg_body(h_hbm, src_hbm, dst_hbm, zf_hbm, agg_out,
                 src_i, dst_i, rows, sbuf, agg_sh, gs0, gs1, is0, is1):
    gsems = (gs0, gs1)
    isems = (is0, is1)
    c = lax.axis_index("c")
    s = lax.axis_index("s")
    wid = s * _NC + c
    wbase = wid * (_E // _NW)
    tb = _NS * _RPT

    def idx_s(i):
        return src_hbm.at[pl.ds(wbase + i * _C, _C)]

    def idx_d(i):
        return dst_hbm.at[pl.ds(wbase + i * _C, _C)]

    def pref(i, b4, sem):
        pltpu.async_copy(idx_s(i), src_i.at[b4], sem)
        pltpu.async_copy(idx_d(i), dst_i.at[b4], sem)

    def wait_i(i, b4, sem):
        pltpu.make_async_copy(idx_s(i), src_i.at[b4], sem).wait()
        pltpu.make_async_copy(idx_d(i), dst_i.at[b4], sem).wait()

    def gath(rb, b4, sem):
        pltpu.async_copy(h_hbm.at[src_i.at[b4]], rows.at[rb], sem)

    def wait_g(rb, b4, sem):
        pltpu.make_async_copy(h_hbm.at[src_i.at[b4]], rows.at[rb], sem).wait()

    def scat(rb, b4):
        pltpu.sync_copy(rows.at[rb], agg_sh.at[dst_i.at[b4]], add=True)

    pltpu.sync_copy(zf_hbm, sbuf)
    for k in range(_RPT // _SB):
        pltpu.sync_copy(sbuf, agg_sh.at[pl.ds(s * _RPT + k * _SB, _SB)])

    @pl.when(s == 0)
    def _():
        pltpu.sync_copy(sbuf.at[pl.ds(0, _RTAIL)], agg_sh.at[pl.ds(tb, _RTAIL)])

    pltpu.sync_copy(idx_s(0), src_i.at[0])
    pltpu.sync_copy(idx_d(0), dst_i.at[0])
    pref(1, 1, isems[1])
    gath(0, 0, gsems[0])
    plsc.subcore_barrier()

    def quad(j, carry):
        for k in range(4):
            i = 4 * j + k
            b4, n4, p4 = k, (k + 1) % 4, (k + 2) % 4
            br, nbr = k % 2, (k + 1) % 2
            wait_i(i + 1, n4, isems[nbr])
            gath(nbr, n4, gsems[nbr])

            @pl.when(i + 2 < _TPW)
            def _():
                pref(i + 2, p4, isems[br])

            wait_g(br, b4, gsems[br])
            scat(br, b4)
        return carry

    lax.fori_loop(0, _TPW // 4, quad, 0)
    wait_g(0, 0, gsems[0])
    scat(0, 0)
    plsc.subcore_barrier()

    for k in range(_RPT // _SB):
        off = s * _RPT + k * _SB
        pltpu.sync_copy(agg_sh.at[pl.ds(off, _SB)], sbuf)
        pltpu.sync_copy(sbuf, agg_out.at[c].at[pl.ds(off, _SB)])

    @pl.when(s == 0)
    def _():
        pltpu.sync_copy(agg_sh.at[pl.ds(tb, _RTAIL)], sbuf.at[pl.ds(0, _RTAIL)])
        pltpu.sync_copy(sbuf.at[pl.ds(0, _RTAIL)],
                        agg_out.at[c].at[pl.ds(tb, _RTAIL)])


_sc_agg = pl.kernel(
    _sc_agg_body,
    out_type=[jax.ShapeDtypeStruct((_NC, _N, _D), jnp.float32)],
    mesh=_mesh,
    scratch_types=[
        pltpu.VMEM((4, _C), jnp.int32),
        pltpu.VMEM((4, _C), jnp.int32),
        pltpu.VMEM((2, _C, _D), jnp.float32),
        pltpu.VMEM((_SB, _D), jnp.float32),
        pltpu.VMEM_SHARED((_N, _D), jnp.float32),
        pltpu.SemaphoreType.DMA,
        pltpu.SemaphoreType.DMA,
        pltpu.SemaphoreType.DMA,
        pltpu.SemaphoreType.DMA,
    ],
)


def _sc_deg_body(dst_hbm, zf_hbm, ones_hbm, deg_out,
                 dst_i, ones_v, sbuf, deg_sh, ss0, ss1):
    ssems = (ss0, ss1)
    c = lax.axis_index("c")
    s = lax.axis_index("s")
    wid = s * _NC + c
    wbase = wid * (_E // _NW)
    tb = _NS * _RPT

    pltpu.sync_copy(zf_hbm, sbuf)
    for k in range(_RPT // _SB):
        pltpu.sync_copy(sbuf, deg_sh.at[pl.ds(s * _RPT + k * _SB, _SB)])
    pltpu.sync_copy(ones_hbm, ones_v)

    @pl.when(s == 0)
    def _():
        pltpu.sync_copy(sbuf.at[pl.ds(0, _RTAIL)], deg_sh.at[pl.ds(tb, _RTAIL)])

    def stage(i, b):
        pltpu.sync_copy(dst_hbm.at[pl.ds(wbase + i * _C, _C)], dst_i.at[b])

    def scat(b):
        pltpu.async_copy(ones_v, deg_sh.at[dst_i.at[b]], ssems[b], add=True)

    def wait_s(b):
        pltpu.make_async_copy(ones_v, deg_sh.at[dst_i.at[b]], ssems[b]).wait()

    stage(0, 0)
    plsc.subcore_barrier()

    scat(0); stage(1, 1)
    scat(1); wait_s(0); stage(2, 0)

    def duo(j, carry):
        for k in range(2):
            b = k
            i = 2 + 2 * j + k
            scat(b)
            wait_s(1 - b)
            stage(i + 1, 1 - b)
        return carry

    lax.fori_loop(0, (_TPW - 3) // 2, duo, 0)
    scat(0); wait_s(1)
    wait_s(0)
    plsc.subcore_barrier()

    for k in range(_RPT // _SB):
        off = s * _RPT + k * _SB
        pltpu.sync_copy(deg_sh.at[pl.ds(off, _SB)], sbuf)
        pltpu.sync_copy(sbuf, deg_out.at[c].at[pl.ds(off, _SB)])

    @pl.when(s == 0)
    def _():
        pltpu.sync_copy(deg_sh.at[pl.ds(tb, _RTAIL)], sbuf.at[pl.ds(0, _RTAIL)])
        pltpu.sync_copy(sbuf.at[pl.ds(0, _RTAIL)],
                        deg_out.at[c].at[pl.ds(tb, _RTAIL)])


_sc_deg = pl.kernel(
    _sc_deg_body,
    out_type=[jax.ShapeDtypeStruct((_NC, _N, _D), jnp.float32)],
    mesh=_mesh,
    scratch_types=[
        pltpu.VMEM((2, _C), jnp.int32),
        pltpu.VMEM((_C, _D), jnp.float32),
        pltpu.VMEM((_SB, _D), jnp.float32),
        pltpu.VMEM_SHARED((_N, _D), jnp.float32),
        pltpu.SemaphoreType.DMA,
        pltpu.SemaphoreType.DMA,
    ],
)


def _mm_t(a, w):
    return lax.dot_general(a, w, (((1,), (1,)), ((), ())),
                           preferred_element_type=jnp.float32)


def _tc_bn_body(agg_ref, deg_ref, h_ref, wn_ref, ws_ref, b_ref, g_ref,
                be_ref, out_ref):
    dd = deg_ref[0] + deg_ref[1]
    invd = 1.0 / jnp.maximum(dd[:, 0:1], 1.0)
    agg = (agg_ref[0] + agg_ref[1]) * invd
    y = _mm_t(agg, wn_ref[...]) + _mm_t(h_ref[...], ws_ref[...]) + b_ref[...]
    m = jnp.mean(y, axis=0, keepdims=True)
    v = jnp.mean((y - m) ** 2, axis=0, keepdims=True)
    yn = (y - m) * lax.rsqrt(v + 1e-5) * g_ref[...] + be_ref[...]
    out_ref[...] = jnp.where(yn >= 0, yn, 0.1 * yn)


def _tc_final_body(agg_ref, deg_ref, h_ref, wn_ref, ws_ref, b_ref, out_ref):
    dd = deg_ref[0] + deg_ref[1]
    invd = 1.0 / jnp.maximum(dd[:, 0:1], 1.0)
    agg = (agg_ref[0] + agg_ref[1]) * invd
    out_ref[...] = (_mm_t(agg, wn_ref[...]) + _mm_t(h_ref[...], ws_ref[...])
                    + b_ref[...])


def _tc_bn(agg, deg, h, wn, ws, b, g, be):
    return pl.pallas_call(
        _tc_bn_body,
        out_shape=jax.ShapeDtypeStruct((_N, _D), jnp.float32),
    )(agg, deg, h, wn, ws, b, g, be)


def _tc_final(agg, deg, h, wn, ws, b):
    return pl.pallas_call(
        _tc_final_body,
        out_shape=jax.ShapeDtypeStruct((_N, _D), jnp.float32),
    )(agg, deg, h, wn, ws, b)


def kernel(x, edge_index, W1n, W1s, b1, g1, be1, W2n, W2s, b2, g2, be2,
           W3n, W3s, b3):
    src = edge_index[0].astype(jnp.int32)
    dst = edge_index[1].astype(jnp.int32)
    zf = jnp.zeros((_SB, _D), jnp.float32)
    ones = jnp.ones((_C, _D), jnp.float32)
    b1r, g1r, be1r = b1.reshape(1, -1), g1.reshape(1, -1), be1.reshape(1, -1)
    b2r, g2r, be2r = b2.reshape(1, -1), g2.reshape(1, -1), be2.reshape(1, -1)
    b3r = b3.reshape(1, -1)

    (deg,) = _sc_deg(dst, zf, ones)
    (agg1,) = _sc_agg(x, src, dst, zf)
    h1 = _tc_bn(agg1, deg, x, W1n, W1s, b1r, g1r, be1r)
    (agg2,) = _sc_agg(h1, src, dst, zf)
    h2 = _tc_bn(agg2, deg, h1, W2n, W2s, b2r, g2r, be2r)
    (agg3,) = _sc_agg(h2, src, dst, zf)
    return _tc_final(agg3, deg, h2, W3n, W3s, b3r)


# speedup vs baseline: 1.8343x; 1.0029x over previous
import jax
import jax.numpy as jnp
from jax import lax
from jax.experimental import pallas as pl
from jax.experimental.pallas import tpu as pltpu
from jax.experimental.pallas import tpu_sc as plsc

_N = 10000
_E = 320000
_D = 128
_NC = 2
_NS = 16
_NW = _NC * _NS
_C = 80
_T = _E // _C
_TPW = _T // _NW
_RPT = (_N // _NS) // 8 * 8
_RTAIL = _N - _NS * _RPT
_SB = 208
_DW = 16

_mesh = plsc.VectorSubcoreMesh(core_axis_name="c", subcore_axis_name="s")


def _sc_agg_body(h_hbm, src_hbm, dst_hbm, zf_hbm, agg_out,
                 src_i, dst_i, rows, sbuf, agg_sh, gs0, gs1, is0, is1):
    gsems = (gs0, gs1)
    isems = (is0, is1)
    c = lax.axis_index("c")
    s = lax.axis_index("s")
    wid = s * _NC + c
    wbase = wid * (_E // _NW)
    tb = _NS * _RPT

    def idx_s(i):
        return src_hbm.at[pl.ds(wbase + i * _C, _C)]

    def idx_d(i):
        return dst_hbm.at[pl.ds(wbase + i * _C, _C)]

    def pref(i, b4, sem):
        pltpu.async_copy(idx_s(i), src_i.at[b4], sem)
        pltpu.async_copy(idx_d(i), dst_i.at[b4], sem)

    def wait_i(i, b4, sem):
        pltpu.make_async_copy(idx_s(i), src_i.at[b4], sem).wait()
        pltpu.make_async_copy(idx_d(i), dst_i.at[b4], sem).wait()

    def gath(rb, b4, sem):
        pltpu.async_copy(h_hbm.at[src_i.at[b4]], rows.at[rb], sem)

    def wait_g(rb, b4, sem):
        pltpu.make_async_copy(h_hbm.at[src_i.at[b4]], rows.at[rb], sem).wait()

    def scat(rb, b4):
        pltpu.sync_copy(rows.at[rb], agg_sh.at[dst_i.at[b4]], add=True)

    pltpu.sync_copy(zf_hbm, sbuf)
    for k in range(_RPT // _SB):
        pltpu.sync_copy(sbuf, agg_sh.at[pl.ds(s * _RPT + k * _SB, _SB)])

    @pl.when(s == 0)
    def _():
        pltpu.sync_copy(sbuf.at[pl.ds(0, _RTAIL)], agg_sh.at[pl.ds(tb, _RTAIL)])

    pltpu.sync_copy(idx_s(0), src_i.at[0])
    pltpu.sync_copy(idx_d(0), dst_i.at[0])
    pref(1, 1, isems[1])
    gath(0, 0, gsems[0])
    plsc.subcore_barrier()

    def quad(j, carry):
        for k in range(4):
            i = 4 * j + k
            b4, n4, p4 = k, (k + 1) % 4, (k + 2) % 4
            br, nbr = k % 2, (k + 1) % 2
            wait_i(i + 1, n4, isems[nbr])
            gath(nbr, n4, gsems[nbr])

            @pl.when(i + 2 < _TPW)
            def _():
                pref(i + 2, p4, isems[br])

            wait_g(br, b4, gsems[br])
            scat(br, b4)
        return carry

    lax.fori_loop(0, _TPW // 4, quad, 0)
    wait_g(0, 0, gsems[0])
    scat(0, 0)
    plsc.subcore_barrier()

    for k in range(_RPT // _SB):
        off = s * _RPT + k * _SB
        pltpu.sync_copy(agg_sh.at[pl.ds(off, _SB)], sbuf)
        pltpu.sync_copy(sbuf, agg_out.at[c].at[pl.ds(off, _SB)])

    @pl.when(s == 0)
    def _():
        pltpu.sync_copy(agg_sh.at[pl.ds(tb, _RTAIL)], sbuf.at[pl.ds(0, _RTAIL)])
        pltpu.sync_copy(sbuf.at[pl.ds(0, _RTAIL)],
                        agg_out.at[c].at[pl.ds(tb, _RTAIL)])


_sc_agg = pl.kernel(
    _sc_agg_body,
    out_type=[jax.ShapeDtypeStruct((_NC, _N, _D), jnp.float32)],
    mesh=_mesh,
    scratch_types=[
        pltpu.VMEM((4, _C), jnp.int32),
        pltpu.VMEM((4, _C), jnp.int32),
        pltpu.VMEM((2, _C, _D), jnp.float32),
        pltpu.VMEM((_SB, _D), jnp.float32),
        pltpu.VMEM_SHARED((_N, _D), jnp.float32),
        pltpu.SemaphoreType.DMA,
        pltpu.SemaphoreType.DMA,
        pltpu.SemaphoreType.DMA,
        pltpu.SemaphoreType.DMA,
    ],
)


def _sc_deg_body(dst_hbm, zf_hbm, ones_hbm, deg_out,
                 dst_i, ones_v, sbuf, deg_sh, ss0, ss1, is0, is1):
    ssems = (ss0, ss1)
    isems = (is0, is1)
    c = lax.axis_index("c")
    s = lax.axis_index("s")
    wid = s * _NC + c
    wbase = wid * (_E // _NW)
    tb = _NS * _RPT

    def idx_d(i):
        return dst_hbm.at[pl.ds(wbase + i * _C, _C)]

    pltpu.sync_copy(zf_hbm, sbuf)
    for k in range(_RPT // _SB):
        pltpu.sync_copy(sbuf, deg_sh.at[pl.ds(s * _RPT + k * _SB, _SB)])
    pltpu.sync_copy(ones_hbm, ones_v)

    @pl.when(s == 0)
    def _():
        pltpu.sync_copy(sbuf.at[pl.ds(0, _RTAIL)], deg_sh.at[pl.ds(tb, _RTAIL)])

    def pref(i, b4, sem):
        pltpu.async_copy(idx_d(i), dst_i.at[b4], sem)

    def wait_i(i, b4, sem):
        pltpu.make_async_copy(idx_d(i), dst_i.at[b4], sem).wait()

    def scat(b4, sem):
        pltpu.async_copy(ones_v, deg_sh.at[dst_i.at[b4]], sem, add=True)

    def wait_s(b4, sem):
        pltpu.make_async_copy(ones_v, deg_sh.at[dst_i.at[b4]], sem).wait()

    pltpu.sync_copy(idx_d(0), dst_i.at[0])
    pref(1, 1, isems[1])
    plsc.subcore_barrier()

    scat(0, ssems[0])

    def quad(j, carry):
        for k in range(4):
            i = 4 * j + k
            b4, n4, p4 = k, (k + 1) % 4, (k + 2) % 4
            br, nbr = k % 2, (k + 1) % 2
            wait_i(i + 1, n4, isems[nbr])
            scat(n4, ssems[nbr])
            wait_s(b4, ssems[br])

            @pl.when(i + 2 < _TPW)
            def _():
                pref(i + 2, p4, isems[br])
        return carry

    lax.fori_loop(0, _TPW // 4, quad, 0)
    wait_s(0, ssems[0])
    plsc.subcore_barrier()

    for k in range(_RPT // _SB):
        off = s * _RPT + k * _SB
        pltpu.sync_copy(deg_sh.at[pl.ds(off, _SB)], sbuf)
        pltpu.sync_copy(sbuf, deg_out.at[c].at[pl.ds(off, _SB)])

    @pl.when(s == 0)
    def _():
        pltpu.sync_copy(deg_sh.at[pl.ds(tb, _RTAIL)], sbuf.at[pl.ds(0, _RTAIL)])
        pltpu.sync_copy(sbuf.at[pl.ds(0, _RTAIL)],
                        deg_out.at[c].at[pl.ds(tb, _RTAIL)])


_sc_deg = pl.kernel(
    _sc_deg_body,
    out_type=[jax.ShapeDtypeStruct((_NC, _N, _D), jnp.float32)],
    mesh=_mesh,
    scratch_types=[
        pltpu.VMEM((4, _C), jnp.int32),
        pltpu.VMEM((_C, _D), jnp.float32),
        pltpu.VMEM((_SB, _D), jnp.float32),
        pltpu.VMEM_SHARED((_N, _D), jnp.float32),
        pltpu.SemaphoreType.DMA,
        pltpu.SemaphoreType.DMA,
        pltpu.SemaphoreType.DMA,
        pltpu.SemaphoreType.DMA,
    ],
)


def _mm_t(a, w):
    return lax.dot_general(a, w, (((1,), (1,)), ((), ())),
                           preferred_element_type=jnp.float32)


def _tc_bn_body(agg_ref, deg_ref, h_ref, wn_ref, ws_ref, b_ref, g_ref,
                be_ref, out_ref):
    dd = deg_ref[0] + deg_ref[1]
    invd = 1.0 / jnp.maximum(dd[:, 0:1], 1.0)
    agg = (agg_ref[0] + agg_ref[1]) * invd
    y = _mm_t(agg, wn_ref[...]) + _mm_t(h_ref[...], ws_ref[...]) + b_ref[...]
    m = jnp.mean(y, axis=0, keepdims=True)
    v = jnp.mean((y - m) ** 2, axis=0, keepdims=True)
    yn = (y - m) * lax.rsqrt(v + 1e-5) * g_ref[...] + be_ref[...]
    out_ref[...] = jnp.where(yn >= 0, yn, 0.1 * yn)


def _tc_final_body(agg_ref, deg_ref, h_ref, wn_ref, ws_ref, b_ref, out_ref):
    dd = deg_ref[0] + deg_ref[1]
    invd = 1.0 / jnp.maximum(dd[:, 0:1], 1.0)
    agg = (agg_ref[0] + agg_ref[1]) * invd
    out_ref[...] = (_mm_t(agg, wn_ref[...]) + _mm_t(h_ref[...], ws_ref[...])
                    + b_ref[...])


def _tc_bn(agg, deg, h, wn, ws, b, g, be):
    return pl.pallas_call(
        _tc_bn_body,
        out_shape=jax.ShapeDtypeStruct((_N, _D), jnp.float32),
    )(agg, deg, h, wn, ws, b, g, be)


def _tc_final(agg, deg, h, wn, ws, b):
    return pl.pallas_call(
        _tc_final_body,
        out_shape=jax.ShapeDtypeStruct((_N, _D), jnp.float32),
    )(agg, deg, h, wn, ws, b)


def kernel(x, edge_index, W1n, W1s, b1, g1, be1, W2n, W2s, b2, g2, be2,
           W3n, W3s, b3):
    src = edge_index[0].astype(jnp.int32)
    dst = edge_index[1].astype(jnp.int32)
    zf = jnp.zeros((_SB, _D), jnp.float32)
    ones = jnp.ones((_C, _D), jnp.float32)
    b1r, g1r, be1r = b1.reshape(1, -1), g1.reshape(1, -1), be1.reshape(1, -1)
    b2r, g2r, be2r = b2.reshape(1, -1), g2.reshape(1, -1), be2.reshape(1, -1)
    b3r = b3.reshape(1, -1)

    (deg,) = _sc_deg(dst, zf, ones)
    (agg1,) = _sc_agg(x, src, dst, zf)
    h1 = _tc_bn(agg1, deg, x, W1n, W1s, b1r, g1r, be1r)
    (agg2,) = _sc_agg(h1, src, dst, zf)
    h2 = _tc_bn(agg2, deg, h1, W2n, W2s, b2r, g2r, be2r)
    (agg3,) = _sc_agg(h2, src, dst, zf)
    return _tc_final(agg3, deg, h2, W3n, W3s, b3r)
